# baseline (speedup 1.0000x reference)
import functools

import jax
import jax.numpy as jnp
from jax.experimental import pallas as pl
from jax.experimental.pallas import tpu as pltpu
from jax.experimental.pallas import tpu_sc as plsc

EMBED_DIM = 768
HIDDEN_DIM = 768
NUM_EXPERTS = 8
TOP_K = 2
T_TOKENS = 2048

_NC, _NS, _L = 2, 16, 16
_NW = _NC * _NS
_CHUNK = T_TOKENS // _NW


def _logits_kernel(wg_ref, x_ref, lt_ref):
    lt_ref[...] = jax.lax.dot_general(
        wg_ref[...], x_ref[...], (((1,), (1,)), ((), ())),
        preferred_element_type=jnp.float32,
    )


def _sc_router(lt_hbm, wcomb_hbm, fp_hbm, pp_hbm, lg_v, wc_v, fp_v, pp_v):
    E = NUM_EXPERTS
    wid = jax.lax.axis_index("s") * _NC + jax.lax.axis_index("c")
    base = wid * _CHUNK
    for e in range(E):
        pltpu.sync_copy(lt_hbm.at[e, pl.ds(base, _CHUNK)],
                        lg_v.at[pl.ds(e * _CHUNK, _CHUNK)])

    f_acc = [jnp.zeros((_L,), jnp.float32) for _ in range(E)]
    p_acc = [jnp.zeros((_L,), jnp.float32) for _ in range(E)]

    for j in range(_CHUNK // _L):
        l = [lg_v[pl.ds(e * _CHUNK + j * _L, _L)] for e in range(E)]
        m = l[0]
        for e in range(1, E):
            m = jnp.maximum(m, l[e])
        ex = [jnp.exp(l[e] - m) for e in range(E)]
        s = ex[0]
        for e in range(1, E):
            s = s + ex[e]
        inv = jnp.float32(1.0) / s
        probs = [ex[e] * inv for e in range(E)]

        big = jnp.full((_L,), E, jnp.int32)
        m1 = probs[0]
        for e in range(1, E):
            m1 = jnp.maximum(m1, probs[e])
        i1 = big
        for e in range(E):
            ev = jnp.full((_L,), e, jnp.int32)
            i1 = jnp.minimum(i1, jnp.where(probs[e] == m1, ev, big))
        neg = jnp.full((_L,), -jnp.inf, jnp.float32)
        masked = []
        for e in range(E):
            ev = jnp.full((_L,), e, jnp.int32)
            masked.append(jnp.where(ev == i1, neg, probs[e]))
        m2 = masked[0]
        for e in range(1, E):
            m2 = jnp.maximum(m2, masked[e])
        i2 = big
        for e in range(E):
            ev = jnp.full((_L,), e, jnp.int32)
            i2 = jnp.minimum(i2, jnp.where(masked[e] == m2, ev, big))

        denom = m1 + m2
        g1 = m1 / denom
        g2 = m2 / denom
        zero = jnp.zeros((_L,), jnp.float32)
        one = jnp.full((_L,), 1.0, jnp.float32)
        for e in range(E):
            ev = jnp.full((_L,), e, jnp.int32)
            sel1 = ev == i1
            sel2 = ev == i2
            wc_v[pl.ds(e * _CHUNK + j * _L, _L)] = (
                jnp.where(sel1, g1, zero) + jnp.where(sel2, g2, zero))
            f_acc[e] = f_acc[e] + jnp.where(sel1, one, zero) + jnp.where(sel2, one, zero)
            p_acc[e] = p_acc[e] + probs[e]

    for e in range(E):
        fp_v[pl.ds(e * _L, _L)] = f_acc[e]
        pp_v[pl.ds(e * _L, _L)] = p_acc[e]
        pltpu.sync_copy(wc_v.at[pl.ds(e * _CHUNK, _CHUNK)],
                        wcomb_hbm.at[e, pl.ds(base, _CHUNK)])
    pltpu.sync_copy(fp_v, fp_hbm.at[wid])
    pltpu.sync_copy(pp_v, pp_hbm.at[wid])


def _ffn_kernel(x_ref, w1_ref, b1_ref, w2_ref, b2_ref, wcomb_ref, fp_ref, pp_ref,
                out_ref, aux_ref, xb16_ref):
    e = pl.program_id(0)

    @pl.when(e == 0)
    def _prep():
        xb16_ref[...] = x_ref[...].astype(jnp.bfloat16)
        fp = fp_ref[...]
        pp = pp_ref[...]
        aux = jnp.float32(0.0)
        for ee in range(NUM_EXPERTS):
            fe = jnp.sum(fp[:, ee * _L:(ee + 1) * _L])
            pe = jnp.sum(pp[:, ee * _L:(ee + 1) * _L])
            aux = aux + fe * pe
        aux = aux * jnp.float32(NUM_EXPERTS) / jnp.float32(T_TOKENS * T_TOKENS)
        aux_ref[...] = aux.reshape(1, 1)

    xb = xb16_ref[...]
    w1 = w1_ref[0].astype(jnp.bfloat16)
    h = jax.lax.dot_general(
        xb, w1, (((1,), (1,)), ((), ())), preferred_element_type=jnp.float32
    ) + b1_ref[0]
    h = h * jax.nn.sigmoid(h)
    w2 = w2_ref[0].astype(jnp.bfloat16)
    eo = jax.lax.dot_general(
        h.astype(jnp.bfloat16), w2, (((1,), (1,)), ((), ())),
        preferred_element_type=jnp.float32,
    ) + b2_ref[0]
    wcol = jnp.transpose(wcomb_ref[pl.ds(e, 1), :])

    @pl.when(e == 0)
    def _first():
        out_ref[...] = wcol * eo

    @pl.when(e > 0)
    def _acc():
        out_ref[...] += wcol * eo


def kernel(x, Wg, W1, b1, W2, b2):
    Bq, Sq, D = x.shape
    T = Bq * Sq
    E = NUM_EXPERTS
    H = HIDDEN_DIM
    xf = x.reshape(T, D)

    logits_t = pl.pallas_call(
        _logits_kernel,
        out_shape=jax.ShapeDtypeStruct((E, T), jnp.float32),
    )(Wg, xf)

    mesh = plsc.VectorSubcoreMesh(core_axis_name="c", subcore_axis_name="s")
    sc_router = pl.kernel(
        _sc_router,
        mesh=mesh,
        out_type=(
            jax.ShapeDtypeStruct((E, T), jnp.float32),
            jax.ShapeDtypeStruct((_NW, E * _L), jnp.float32),
            jax.ShapeDtypeStruct((_NW, E * _L), jnp.float32),
        ),
        scratch_types=[
            pltpu.VMEM((E * _CHUNK,), jnp.float32),
            pltpu.VMEM((E * _CHUNK,), jnp.float32),
            pltpu.VMEM((E * _L,), jnp.float32),
            pltpu.VMEM((E * _L,), jnp.float32),
        ],
    )
    wcomb_t, f_part, p_part = sc_router(logits_t)

    out, aux = pl.pallas_call(
        _ffn_kernel,
        grid=(E,),
        in_specs=[
            pl.BlockSpec((T, D), lambda e: (0, 0)),
            pl.BlockSpec((1, H, D), lambda e: (e, 0, 0)),
            pl.BlockSpec((1, 1, H), lambda e: (e, 0, 0)),
            pl.BlockSpec((1, D, H), lambda e: (e, 0, 0)),
            pl.BlockSpec((1, 1, D), lambda e: (e, 0, 0)),
            pl.BlockSpec((E, T), lambda e: (0, 0)),
            pl.BlockSpec((_NW, E * _L), lambda e: (0, 0)),
            pl.BlockSpec((_NW, E * _L), lambda e: (0, 0)),
        ],
        out_specs=(
            pl.BlockSpec((T, D), lambda e: (0, 0)),
            pl.BlockSpec((1, 1), lambda e: (0, 0)),
        ),
        out_shape=(
            jax.ShapeDtypeStruct((T, D), jnp.float32),
            jax.ShapeDtypeStruct((1, 1), jnp.float32),
        ),
        scratch_shapes=[
            pltpu.VMEM((T, D), jnp.bfloat16),
        ],
    )(xf, W1, b1.reshape(E, 1, H), W2, b2.reshape(E, 1, D), wcomb_t, f_part, p_part)

    return out.reshape(Bq, Sq, D), aux.reshape(())


# speedup vs baseline: 1.0801x; 1.0801x over previous
import jax
import jax.numpy as jnp
from jax.experimental import pallas as pl

EMBED_DIM = 768
HIDDEN_DIM = 768
NUM_EXPERTS = 8
TOP_K = 2


def _router_kernel(x_ref, wg_ref, wcomb_ref, aux_ref):
    x = x_ref[...]
    wg = wg_ref[...]
    logits = jax.lax.dot_general(
        x, wg, (((1,), (1,)), ((), ())), preferred_element_type=jnp.float32
    )
    m = jnp.max(logits, axis=-1, keepdims=True)
    ex = jnp.exp(logits - m)
    probs = ex / jnp.sum(ex, axis=-1, keepdims=True)

    T, E = probs.shape
    idx = jax.lax.broadcasted_iota(jnp.int32, (T, E), 1)
    big = jnp.int32(E)
    m1 = jnp.max(probs, axis=-1, keepdims=True)
    i1 = jnp.min(jnp.where(probs == m1, idx, big), axis=-1, keepdims=True)
    masked = jnp.where(idx == i1, -jnp.inf, probs)
    m2 = jnp.max(masked, axis=-1, keepdims=True)
    i2 = jnp.min(jnp.where(masked == m2, idx, big), axis=-1, keepdims=True)

    denom = m1 + m2
    g1 = m1 / denom
    g2 = m2 / denom

    onehot1 = (idx == i1).astype(jnp.float32)
    onehot2 = (idx == i2).astype(jnp.float32)
    wcomb = g1 * onehot1 + g2 * onehot2
    wcomb_ref[...] = jnp.transpose(wcomb)

    f = jnp.sum(onehot1 + onehot2, axis=0) / jnp.float32(T)
    p = jnp.sum(probs, axis=0) / jnp.float32(T)
    aux_ref[...] = (jnp.float32(NUM_EXPERTS) * jnp.sum(f * p)).reshape(1, 1)


def _ffn_kernel(x_ref, w1_ref, b1_ref, w2_ref, b2_ref, wt_ref, out_ref):
    e = pl.program_id(1)

    @pl.when(e == 0)
    def _init():
        out_ref[...] = jnp.zeros_like(out_ref)

    xb = x_ref[...].astype(jnp.bfloat16)
    w1 = w1_ref[0].astype(jnp.bfloat16)
    h = jax.lax.dot_general(
        xb, w1, (((1,), (1,)), ((), ())), preferred_element_type=jnp.float32
    ) + b1_ref[0]
    h = h * jax.nn.sigmoid(h)
    w2 = w2_ref[0].astype(jnp.bfloat16)
    eo = jax.lax.dot_general(
        h.astype(jnp.bfloat16), w2, (((1,), (1,)), ((), ())),
        preferred_element_type=jnp.float32,
    ) + b2_ref[0]
    wcol = jnp.transpose(wt_ref[pl.ds(e, 1), :])
    out_ref[...] += wcol * eo


def kernel(x, Wg, W1, b1, W2, b2):
    Bq, Sq, D = x.shape
    T = Bq * Sq
    E = NUM_EXPERTS
    H = HIDDEN_DIM
    xf = x.reshape(T, D)

    wcomb_t, aux = pl.pallas_call(
        _router_kernel,
        out_shape=(
            jax.ShapeDtypeStruct((E, T), jnp.float32),
            jax.ShapeDtypeStruct((1, 1), jnp.float32),
        ),
    )(xf, Wg)

    BT = 512
    n_t = T // BT
    out = pl.pallas_call(
        _ffn_kernel,
        grid=(n_t, E),
        in_specs=[
            pl.BlockSpec((BT, D), lambda t, e: (t, 0)),
            pl.BlockSpec((1, H, D), lambda t, e: (e, 0, 0)),
            pl.BlockSpec((1, 1, H), lambda t, e: (e, 0, 0)),
            pl.BlockSpec((1, D, H), lambda t, e: (e, 0, 0)),
            pl.BlockSpec((1, 1, D), lambda t, e: (e, 0, 0)),
            pl.BlockSpec((NUM_EXPERTS, BT), lambda t, e: (0, t)),
        ],
        out_specs=pl.BlockSpec((BT, D), lambda t, e: (t, 0)),
        out_shape=jax.ShapeDtypeStruct((T, D), jnp.float32),
    )(xf, W1, b1.reshape(E, 1, H), W2, b2.reshape(E, 1, D), wcomb_t)

    return out.reshape(Bq, Sq, D), aux.reshape(())


# speedup vs baseline: 1.3598x; 1.2590x over previous
import jax
import jax.numpy as jnp
from jax.experimental import pallas as pl

EMBED_DIM = 768
HIDDEN_DIM = 768
NUM_EXPERTS = 8
TOP_K = 2


def _router_kernel(x_ref, wg_ref, wcomb_ref, aux_ref):
    x = x_ref[...]
    wg = wg_ref[...]
    logits = jax.lax.dot_general(
        x, wg, (((1,), (1,)), ((), ())), preferred_element_type=jnp.float32
    )
    m = jnp.max(logits, axis=-1, keepdims=True)
    ex = jnp.exp(logits - m)
    probs = ex / jnp.sum(ex, axis=-1, keepdims=True)

    T, E = probs.shape
    idx = jax.lax.broadcasted_iota(jnp.int32, (T, E), 1)
    big = jnp.int32(E)
    m1 = jnp.max(probs, axis=-1, keepdims=True)
    i1 = jnp.min(jnp.where(probs == m1, idx, big), axis=-1, keepdims=True)
    masked = jnp.where(idx == i1, -jnp.inf, probs)
    m2 = jnp.max(masked, axis=-1, keepdims=True)
    i2 = jnp.min(jnp.where(masked == m2, idx, big), axis=-1, keepdims=True)

    denom = m1 + m2
    g1 = m1 / denom
    g2 = m2 / denom

    onehot1 = (idx == i1).astype(jnp.float32)
    onehot2 = (idx == i2).astype(jnp.float32)
    wcomb = g1 * onehot1 + g2 * onehot2
    wcomb_ref[...] = jnp.transpose(wcomb)

    f = jnp.sum(onehot1 + onehot2, axis=0) / jnp.float32(T)
    p = jnp.sum(probs, axis=0) / jnp.float32(T)
    aux_ref[...] = (jnp.float32(NUM_EXPERTS) * jnp.sum(f * p)).reshape(1, 1)


def _ffn_kernel(x_ref, w1_ref, b1_ref, w2_ref, b2_ref, wt_ref, out_ref):
    e = pl.program_id(1)

    @pl.when(e == 0)
    def _init():
        out_ref[...] = jnp.zeros_like(out_ref)

    xb = x_ref[...].astype(jnp.bfloat16)
    w1 = w1_ref[0].astype(jnp.bfloat16)
    h = jax.lax.dot_general(
        xb, w1, (((1,), (1,)), ((), ())), preferred_element_type=jnp.float32
    ) + b1_ref[0]
    h = h * jax.nn.sigmoid(h)
    w2 = w2_ref[0].astype(jnp.bfloat16)
    eo = jax.lax.dot_general(
        h.astype(jnp.bfloat16), w2, (((1,), (1,)), ((), ())),
        preferred_element_type=jnp.float32,
    ) + b2_ref[0]
    wcol = jnp.transpose(wt_ref[pl.ds(e, 1), :])
    out_ref[...] += wcol * eo


def kernel(x, Wg, W1, b1, W2, b2):
    Bq, Sq, D = x.shape
    T = Bq * Sq
    E = NUM_EXPERTS
    H = HIDDEN_DIM
    xf = x.reshape(T, D)

    wcomb_t, aux = pl.pallas_call(
        _router_kernel,
        out_shape=(
            jax.ShapeDtypeStruct((E, T), jnp.float32),
            jax.ShapeDtypeStruct((1, 1), jnp.float32),
        ),
    )(xf, Wg)

    BT = 1024
    n_t = T // BT
    out = pl.pallas_call(
        _ffn_kernel,
        grid=(n_t, E),
        in_specs=[
            pl.BlockSpec((BT, D), lambda t, e: (t, 0)),
            pl.BlockSpec((1, H, D), lambda t, e: (e, 0, 0)),
            pl.BlockSpec((1, 1, H), lambda t, e: (e, 0, 0)),
            pl.BlockSpec((1, D, H), lambda t, e: (e, 0, 0)),
            pl.BlockSpec((1, 1, D), lambda t, e: (e, 0, 0)),
            pl.BlockSpec((NUM_EXPERTS, BT), lambda t, e: (0, t)),
        ],
        out_specs=pl.BlockSpec((BT, D), lambda t, e: (t, 0)),
        out_shape=jax.ShapeDtypeStruct((T, D), jnp.float32),
    )(xf, W1, b1.reshape(E, 1, H), W2, b2.reshape(E, 1, D), wcomb_t)

    return out.reshape(Bq, Sq, D), aux.reshape(())
